# Initial kernel scaffold; baseline (speedup 1.0000x reference)
import jax
import jax.numpy as jnp
from jax.experimental import pallas as pl


def kernel(x, table):
    raise NotImplementedError("write your pallas kernel here")


# speedup vs baseline: 1.3070x; 1.3070x over previous
import functools

import jax
import jax.numpy as jnp
from jax import lax
from jax.experimental import pallas as pl
from jax.experimental.pallas import tpu as pltpu
from jax.experimental.pallas import tpu_sc as plsc

NC = 2
NS = 16
NW = NC * NS
BLK = 128


def _emb_kernel(steps, emb, idx_hbm, table_hbm, out_hbm, idx_v, buf, gsem):
    wid = lax.axis_index("s") * NC + lax.axis_index("c")
    pltpu.sync_copy(idx_hbm.at[wid], idx_v)
    obase = wid * steps

    def body(j, carry):
        pltpu.async_copy(table_hbm.at[idx_v.at[j]], buf, gsem).wait()
        pltpu.sync_copy(buf, out_hbm.at[obase + j])
        return carry

    lax.fori_loop(0, steps, body, 0)


def kernel(x, table):
    b, h = x.shape
    v, d = table.shape
    total = b * h
    assert total % (NW * BLK) == 0
    steps = total // (NW * BLK)

    x3 = x.reshape(NW, steps, BLK).astype(jnp.int32)

    mesh = plsc.VectorSubcoreMesh(
        core_axis_name="c", subcore_axis_name="s", num_cores=NC, num_subcores=NS
    )
    run = pl.kernel(
        functools.partial(_emb_kernel, steps, d),
        mesh=mesh,
        out_type=jax.ShapeDtypeStruct((NW * steps, BLK, d), jnp.float32),
        scratch_types=[
            pltpu.VMEM((steps, BLK), jnp.int32),
            pltpu.VMEM((BLK, d), jnp.float32),
            pltpu.SemaphoreType.DMA,
        ],
        compiler_params=pltpu.CompilerParams(use_tc_tiling_on_sc=False),
    )
    out = run(x3, table)
    return out.reshape(b, h, d)


# speedup vs baseline: 1.4970x; 1.1454x over previous
import functools

import jax
import jax.numpy as jnp
from jax import lax
from jax.experimental import pallas as pl
from jax.experimental.pallas import tpu as pltpu
from jax.experimental.pallas import tpu_sc as plsc

NC = 2
NS = 16
NW = NC * NS
BLK = 128
NBUF = 8


def _emb_kernel(steps, emb, idx_hbm, table_hbm, out_hbm, idx_v, bufs, gsems, ssems):
    wid = lax.axis_index("s") * NC + lax.axis_index("c")
    pltpu.sync_copy(idx_hbm.at[wid], idx_v)
    obase = wid * steps
    ngroups = steps // NBUF

    def gather(b, j):
        pltpu.async_copy(table_hbm.at[idx_v.at[j]], bufs.at[b], gsems.at[b])

    def scatter(b, j):
        pltpu.async_copy(bufs.at[b], out_hbm.at[obase + j], ssems.at[b])

    for b in range(NBUF):
        gather(b, b)

    def body(g, carry):
        j0 = g * NBUF
        for b in range(NBUF):
            pltpu.make_async_copy(table_hbm.at[idx_v.at[0]], bufs.at[b], gsems.at[b]).wait()
            scatter(b, j0 + b)
        for b in range(NBUF):
            pltpu.make_async_copy(bufs.at[b], out_hbm.at[0], ssems.at[b]).wait()
            gather(b, j0 + NBUF + b)
        return carry

    lax.fori_loop(0, ngroups - 1, body, 0)

    j0 = (ngroups - 1) * NBUF
    for b in range(NBUF):
        pltpu.make_async_copy(table_hbm.at[idx_v.at[0]], bufs.at[b], gsems.at[b]).wait()
        scatter(b, j0 + b)
    for b in range(NBUF):
        pltpu.make_async_copy(bufs.at[b], out_hbm.at[0], ssems.at[b]).wait()


def kernel(x, table):
    b, h = x.shape
    v, d = table.shape
    total = b * h
    assert total % (NW * BLK) == 0
    steps = total // (NW * BLK)
    assert steps % NBUF == 0

    x3 = x.reshape(NW, steps, BLK).astype(jnp.int32)

    mesh = plsc.VectorSubcoreMesh(
        core_axis_name="c", subcore_axis_name="s", num_cores=NC, num_subcores=NS
    )
    run = pl.kernel(
        functools.partial(_emb_kernel, steps, d),
        mesh=mesh,
        out_type=jax.ShapeDtypeStruct((NW * steps, BLK, d), jnp.float32),
        scratch_types=[
            pltpu.VMEM((steps, BLK), jnp.int32),
            pltpu.VMEM((NBUF, BLK, d), jnp.float32),
            pltpu.SemaphoreType.DMA((NBUF,)),
            pltpu.SemaphoreType.DMA((NBUF,)),
        ],
        compiler_params=pltpu.CompilerParams(use_tc_tiling_on_sc=False),
    )
    out = run(x3, table)
    return out.reshape(b, h, d)


# speedup vs baseline: 1.4993x; 1.0015x over previous
import functools

import jax
import jax.numpy as jnp
from jax import lax
from jax.experimental import pallas as pl
from jax.experimental.pallas import tpu as pltpu
from jax.experimental.pallas import tpu_sc as plsc

NC = 2
NS = 16
NW = NC * NS
NBUF = 8
SPLIT = 104


def _emb_kernel(rows, h, emb, idx_hbm, table_hbm, out_hbm, idx_v, bufs, gsems, ssems):
    wid = lax.axis_index("s") * NC + lax.axis_index("c")
    rbase = wid * rows
    pltpu.sync_copy(idx_hbm.at[pl.ds(rbase, rows)], idx_v)
    ngroups = rows // NBUF
    c1 = h - SPLIT

    def gather(b, r):
        pltpu.async_copy(
            table_hbm.at[idx_v.at[r, pl.ds(0, SPLIT)]],
            bufs.at[b, pl.ds(0, SPLIT)],
            gsems.at[b],
        )
        pltpu.async_copy(
            table_hbm.at[idx_v.at[r, pl.ds(SPLIT, c1)]],
            bufs.at[b, pl.ds(SPLIT, c1)],
            gsems.at[b],
        )

    def gwait(b):
        pltpu.make_async_copy(
            table_hbm.at[pl.ds(0, h)], bufs.at[b], gsems.at[b]
        ).wait()

    def scatter(b, r):
        pltpu.async_copy(bufs.at[b], out_hbm.at[rbase + r], ssems.at[b])

    def swait(b):
        pltpu.make_async_copy(bufs.at[b], out_hbm.at[0], ssems.at[b]).wait()

    for b in range(NBUF):
        gather(b, b)

    def body(g, carry):
        r0 = g * NBUF
        for b in range(NBUF):
            gwait(b)
            scatter(b, r0 + b)
        for b in range(NBUF):
            swait(b)
            gather(b, r0 + NBUF + b)
        return carry

    lax.fori_loop(0, ngroups - 1, body, 0)

    r0 = (ngroups - 1) * NBUF
    for b in range(NBUF):
        gwait(b)
        scatter(b, r0 + b)
    for b in range(NBUF):
        swait(b)


def kernel(x, table):
    b, h = x.shape
    v, d = table.shape
    assert b % NW == 0
    rows = b // NW
    assert rows % NBUF == 0

    mesh = plsc.VectorSubcoreMesh(
        core_axis_name="c", subcore_axis_name="s", num_cores=NC, num_subcores=NS
    )
    run = pl.kernel(
        functools.partial(_emb_kernel, rows, h, d),
        mesh=mesh,
        out_type=jax.ShapeDtypeStruct((b, h, d), jnp.float32),
        scratch_types=[
            pltpu.VMEM((rows, h), jnp.int32),
            pltpu.VMEM((NBUF, h, d), jnp.float32),
            pltpu.SemaphoreType.DMA((NBUF,)),
            pltpu.SemaphoreType.DMA((NBUF,)),
        ],
        compiler_params=pltpu.CompilerParams(use_tc_tiling_on_sc=False),
    )
    return run(x.astype(jnp.int32), table)
